# baseline (speedup 1.0000x reference)
import functools

import jax
import jax.numpy as jnp
from jax import lax
from jax.experimental import pallas as pl
from jax.experimental.pallas import tpu as pltpu
from jax.experimental.pallas import tpu_sc as plsc

_D = 128
_NW = 32
_GC = 240
_GS = 40
_EPAD = _NW * _GC * _GS

_BLK = 2000


def _gate_body(occ_ref, sym_ref, wu_ref, bu_ref, wg1_ref, wg2_ref, bg_ref, out_ref):
    occ = occ_ref[...]
    sym = sym_ref[...]
    u = jnp.dot(sym, wu_ref[...], preferred_element_type=jnp.float32) + bu_ref[...]
    u = jnp.maximum(u, 0.0)
    z = (jnp.dot(occ, wg1_ref[...], preferred_element_type=jnp.float32)
         + jnp.dot(u, wg2_ref[...], preferred_element_type=jnp.float32)
         + bg_ref[...])
    g = jax.nn.sigmoid(z)
    out_ref[...] = g * occ + (1.0 - g) * u


def _copy_body(flat_ref, out_ref):
    out_ref[...] = flat_ref[...]


def _gate_alias_body(occ_ref, sym_ref, wu_ref, bu_ref, wg1_ref, wg2_ref,
                     bg_ref, v0_ref, out_ref):
    del v0_ref
    _gate_body(occ_ref, sym_ref, wu_ref, bu_ref, wg1_ref, wg2_ref, bg_ref,
               out_ref)


def _build_v(flat, occ, sym, Wu, bu, Wg1, Wg2, bg, e, bt):
    d = _D
    nv = e + bt
    row_spec = pl.BlockSpec((_BLK, d), lambda i: (i, 0))
    full_spec = pl.BlockSpec((d, d), lambda i: (0, 0))
    bias_spec = pl.BlockSpec((1, d), lambda i: (0, 0))
    ecols = e // _BLK
    v0 = pl.pallas_call(
        _copy_body,
        grid=(bt // _BLK,),
        in_specs=[row_spec],
        out_specs=pl.BlockSpec((_BLK, d), lambda i, ecols=ecols: (i + ecols, 0)),
        out_shape=jax.ShapeDtypeStruct((nv, d), jnp.float32),
    )(flat)
    return pl.pallas_call(
        _gate_alias_body,
        grid=(ecols,),
        in_specs=[row_spec, row_spec, full_spec, bias_spec, full_spec,
                  full_spec, bias_spec,
                  pl.BlockSpec(memory_space=pl.ANY)],
        out_specs=row_spec,
        out_shape=jax.ShapeDtypeStruct((nv, d), jnp.float32),
        input_output_aliases={7: 0},
    )(occ, sym, Wu, bu.reshape(1, d), Wg1, Wg2, bg.reshape(1, d), v0)


def _gather_two(flat, symbols, fidx_pad, sidx_pad):
    mesh = plsc.VectorSubcoreMesh(core_axis_name="c", subcore_axis_name="s")

    @functools.partial(
        pl.kernel,
        out_type=(jax.ShapeDtypeStruct((_EPAD, _D), jnp.float32),
                  jax.ShapeDtypeStruct((_EPAD, _D), jnp.float32)),
        mesh=mesh,
        scratch_types=[
            pltpu.VMEM((_GC,), jnp.int32),
            pltpu.VMEM((_GC,), jnp.int32),
            pltpu.VMEM((_GC, _D), jnp.float32),
            pltpu.VMEM((_GC, _D), jnp.float32),
            pltpu.SemaphoreType.DMA,
            pltpu.SemaphoreType.DMA,
        ],
    )
    def k(flat_hbm, sym_hbm, fidx_hbm, sidx_hbm, occ_hbm, symo_hbm,
          idx0, idx1, buf0, buf1, sem0, sem1):
        wid = lax.axis_index("s") * 2 + lax.axis_index("c")
        base = wid * (_GC * _GS)
        for tbl, ih, oh in ((flat_hbm, fidx_hbm, occ_hbm),
                            (sym_hbm, sidx_hbm, symo_hbm)):
            pltpu.sync_copy(ih.at[pl.ds(base, _GC)], idx0)
            pltpu.async_copy(tbl.at[idx0], buf0, sem0)
            pltpu.sync_copy(ih.at[pl.ds(base + _GC, _GC)], idx1)
            pltpu.async_copy(tbl.at[idx1], buf1, sem1)

            def pair(g, carry, tbl=tbl, ih=ih, oh=oh):
                for b, (idxb, bufb, semb) in enumerate(
                        ((idx0, buf0, sem0), (idx1, buf1, sem1))):
                    s = 2 * g + b
                    pltpu.make_async_copy(tbl.at[idxb], bufb, semb).wait()
                    pltpu.sync_copy(bufb, oh.at[pl.ds(base + s * _GC, _GC)])

                    @pl.when(s + 2 < _GS)
                    def _():
                        pltpu.sync_copy(ih.at[pl.ds(base + (s + 2) * _GC, _GC)],
                                        idxb)
                        pltpu.async_copy(tbl.at[idxb], bufb, semb)
                return carry

            lax.fori_loop(0, _GS // 2, pair, 0)

    return k(flat, symbols, fidx_pad, sidx_pad)


_WC = 19200
_WP = 14


def _winner(widx_pad, bt, e):
    mesh = plsc.VectorSubcoreMesh(core_axis_name="c", subcore_axis_name="s",
                                  num_cores=1)
    nrow = bt + 8
    q = 15632

    @functools.partial(
        pl.kernel,
        out_type=jax.ShapeDtypeStruct((nrow,), jnp.int32),
        mesh=mesh,
        scratch_types=[
            pltpu.VMEM((q,), jnp.int32),
            pltpu.VMEM((_WC,), jnp.int32),
            pltpu.VMEM((_WC,), jnp.int32),
            pltpu.VMEM((_WC,), jnp.int32),
            pltpu.VMEM((_WC,), jnp.int32),
            pltpu.SemaphoreType.DMA,
        ],
    )
    def k(widx_hbm, ptr_hbm, initb, idxb, keyb, gatb, sidxb, sem):
        wid = lax.axis_index("s")
        iota = lax.iota(jnp.int32, 16)

        ibase = jnp.minimum(wid * q, nrow - q)

        def istep(j, _):
            initb[pl.ds(16 * j, 16)] = ibase + 16 * j + iota + 1
            return 0
        lax.fori_loop(0, q // 16, istep, 0)
        pltpu.sync_copy(initb, ptr_hbm.at[pl.ds(ibase, q)])

        wbase = wid * _WC
        pltpu.sync_copy(widx_hbm.at[pl.ds(wbase, _WC)], idxb)

        def kstep(j, _):
            keyb[pl.ds(16 * j, 16)] = -(wbase + 16 * j + iota)
            return 0
        lax.fori_loop(0, _WC // 16, kstep, 0)

        plsc.subcore_barrier()
        pltpu.async_copy(keyb, ptr_hbm.at[idxb], sem).wait()
        plsc.subcore_barrier()

        def one_pass(p, _):
            pltpu.async_copy(ptr_hbm.at[idxb], gatb, sem).wait()

            def cstep(j, _):
                sl = pl.ds(16 * j, 16)
                sidxb[sl] = jnp.where(keyb[sl] < gatb[sl], idxb[sl], bt)
                return 0
            lax.fori_loop(0, _WC // 16, cstep, 0)
            pltpu.async_copy(keyb, ptr_hbm.at[sidxb], sem).wait()
            plsc.subcore_barrier()
            return 0
        lax.fori_loop(0, _WP, one_pass, 0)

    return k(widx_pad)


_DC = 256
_DL = 32


def _dest(vtab, keytab, e, bt):
    mesh = plsc.VectorSubcoreMesh(core_axis_name="c", subcore_axis_name="s")
    cmax = bt // _DC
    lastbase = bt - _DC

    @functools.partial(
        pl.kernel,
        out_type=jax.ShapeDtypeStruct((bt, _D), jnp.float32),
        mesh=mesh,
        scratch_types=[
            pltpu.VMEM((_DC,), jnp.int32),
            pltpu.VMEM((_DC,), jnp.int32),
            pltpu.VMEM((_DC,), jnp.int32),
            pltpu.VMEM((_DC,), jnp.int32),
            pltpu.VMEM((_DC, _D), jnp.float32),
            pltpu.VMEM((_DC, _D), jnp.float32),
            pltpu.SemaphoreType.DMA,
            pltpu.SemaphoreType.DMA,
        ],
    )
    def k(v_hbm, key_hbm, out_hbm, ptr0, vr0, ptr1, vr1, buf0, buf1,
          sem0, sem1):
        wid = lax.axis_index("s") * 2 + lax.axis_index("c")
        iota16 = lax.iota(jnp.int32, 16)
        del iota16

        def base_of(l):
            c = jnp.minimum(wid + _NW * l, cmax)
            return jnp.minimum(c * _DC, lastbase)

        def load_start(l, ptrb, vrb, bufb, semb):
            bs = base_of(l)
            pltpu.sync_copy(key_hbm.at[pl.ds(bs, _DC)], ptrb)

            def vstep(i, _):
                sl = pl.ds(16 * i, 16)
                key = ptrb[sl]
                vrb[sl] = jnp.where(key <= 0, -key, e + key - 1)
                return 0
            lax.fori_loop(0, _DC // 16, vstep, 0)
            pltpu.async_copy(v_hbm.at[vrb], bufb, semb)

        bufs = ((ptr0, vr0, buf0, sem0), (ptr1, vr1, buf1, sem1))
        load_start(0, *bufs[0])
        load_start(1, *bufs[1])

        def pair(j, carry):
            for bb, (ptrb, vrb, bufb, semb) in enumerate(bufs):
                l = 2 * j + bb
                pltpu.make_async_copy(v_hbm.at[vrb], bufb, semb).wait()
                pltpu.sync_copy(bufb, out_hbm.at[pl.ds(base_of(l), _DC)])

                @pl.when(l + 2 < _DL)
                def _():
                    load_start(l + 2, ptrb, vrb, bufb, semb)
            return carry

        lax.fori_loop(0, _DL // 2, pair, 0)

    return k(vtab, keytab)


def kernel(expressions_encodings, symbols_encodings, expr_idx, token_idx,
           symbol_idx, Wu, bu, Wg, bg):
    b, t, d = expressions_encodings.shape
    e = expr_idx.shape[0]
    flat = expressions_encodings.reshape(b * t, d)
    flat_idx = (t * expr_idx + token_idx).astype(jnp.int32)
    fidx_pad = jnp.pad(flat_idx, (0, _EPAD - e))
    sidx_pad = jnp.pad(symbol_idx.astype(jnp.int32), (0, _EPAD - e))
    occ_p, sym_p = _gather_two(flat, symbols_encodings, fidx_pad, sidx_pad)
    bt = b * t
    vtab = _build_v(flat, occ_p, sym_p, Wu, bu, Wg[:d], Wg[d:], bg, e, bt)
    widx_pad = jnp.concatenate([flat_idx, jnp.full((_EPAD - e,), bt, jnp.int32)])
    keytab = _winner(widx_pad, bt, e)
    out = _dest(vtab, keytab, e, bt)
    return out.reshape(b, t, d)


# speedup vs baseline: 201.0333x; 201.0333x over previous
import functools

import jax
import jax.numpy as jnp
from jax import lax
from jax.experimental import pallas as pl
from jax.experimental.pallas import tpu as pltpu
from jax.experimental.pallas import tpu_sc as plsc

_D = 128
_NW = 32
_GC = 480
_GS = 20
_EPAD = _NW * _GC * _GS

_BLK = 2000


def _gate_body(occ_ref, sym_ref, wu_ref, bu_ref, wg1_ref, wg2_ref, bg_ref, out_ref):
    occ = occ_ref[...]
    sym = sym_ref[...]
    u = jnp.dot(sym, wu_ref[...], preferred_element_type=jnp.float32) + bu_ref[...]
    u = jnp.maximum(u, 0.0)
    z = (jnp.dot(occ, wg1_ref[...], preferred_element_type=jnp.float32)
         + jnp.dot(u, wg2_ref[...], preferred_element_type=jnp.float32)
         + bg_ref[...])
    g = jax.nn.sigmoid(z)
    out_ref[...] = g * occ + (1.0 - g) * u


def _copy_body(flat_ref, out_ref):
    out_ref[...] = flat_ref[...]


def _gate_alias_body(occ_ref, sym_ref, wu_ref, bu_ref, wg1_ref, wg2_ref,
                     bg_ref, v0_ref, out_ref):
    del v0_ref
    _gate_body(occ_ref, sym_ref, wu_ref, bu_ref, wg1_ref, wg2_ref, bg_ref,
               out_ref)


def _build_v(flat, occ, sym, Wu, bu, Wg1, Wg2, bg, e, bt):
    d = _D
    nv = e + bt
    row_spec = pl.BlockSpec((_BLK, d), lambda i: (i, 0))
    full_spec = pl.BlockSpec((d, d), lambda i: (0, 0))
    bias_spec = pl.BlockSpec((1, d), lambda i: (0, 0))
    ecols = e // _BLK
    v0 = pl.pallas_call(
        _copy_body,
        grid=(bt // _BLK,),
        in_specs=[row_spec],
        out_specs=pl.BlockSpec((_BLK, d), lambda i, ecols=ecols: (i + ecols, 0)),
        out_shape=jax.ShapeDtypeStruct((nv, d), jnp.float32),
    )(flat)
    return pl.pallas_call(
        _gate_alias_body,
        grid=(ecols,),
        in_specs=[row_spec, row_spec, full_spec, bias_spec, full_spec,
                  full_spec, bias_spec,
                  pl.BlockSpec(memory_space=pl.ANY)],
        out_specs=row_spec,
        out_shape=jax.ShapeDtypeStruct((nv, d), jnp.float32),
        input_output_aliases={7: 0},
    )(occ, sym, Wu, bu.reshape(1, d), Wg1, Wg2, bg.reshape(1, d), v0)


def _gather_two(flat, symbols, fidx_pad, sidx_pad):
    mesh = plsc.VectorSubcoreMesh(core_axis_name="c", subcore_axis_name="s")

    @functools.partial(
        pl.kernel,
        out_type=(jax.ShapeDtypeStruct((_EPAD, _D), jnp.float32),
                  jax.ShapeDtypeStruct((_EPAD, _D), jnp.float32)),
        mesh=mesh,
        scratch_types=[
            pltpu.VMEM((_GC,), jnp.int32),
            pltpu.VMEM((_GC,), jnp.int32),
            pltpu.VMEM((_GC, _D), jnp.float32),
            pltpu.VMEM((_GC, _D), jnp.float32),
            pltpu.SemaphoreType.DMA,
            pltpu.SemaphoreType.DMA,
        ],
    )
    def k(flat_hbm, sym_hbm, fidx_hbm, sidx_hbm, occ_hbm, symo_hbm,
          idx0, idx1, buf0, buf1, sem0, sem1):
        wid = lax.axis_index("s") * 2 + lax.axis_index("c")
        base = wid * (_GC * _GS)
        for tbl, ih, oh in ((flat_hbm, fidx_hbm, occ_hbm),
                            (sym_hbm, sidx_hbm, symo_hbm)):
            pltpu.sync_copy(ih.at[pl.ds(base, _GC)], idx0)
            pltpu.async_copy(tbl.at[idx0], buf0, sem0)
            pltpu.sync_copy(ih.at[pl.ds(base + _GC, _GC)], idx1)
            pltpu.async_copy(tbl.at[idx1], buf1, sem1)

            def pair(g, carry, tbl=tbl, ih=ih, oh=oh):
                for b, (idxb, bufb, semb) in enumerate(
                        ((idx0, buf0, sem0), (idx1, buf1, sem1))):
                    s = 2 * g + b
                    pltpu.make_async_copy(tbl.at[idxb], bufb, semb).wait()
                    pltpu.sync_copy(bufb, oh.at[pl.ds(base + s * _GC, _GC)])

                    @pl.when(s + 2 < _GS)
                    def _():
                        pltpu.sync_copy(ih.at[pl.ds(base + (s + 2) * _GC, _GC)],
                                        idxb)
                        pltpu.async_copy(tbl.at[idxb], bufb, semb)
                return carry

            lax.fori_loop(0, _GS // 2, pair, 0)

    return k(flat, symbols, fidx_pad, sidx_pad)


_WC = 19200
_WP = 12
_WCAP = 4800


def _winner(widx_pad, bt, e):
    mesh = plsc.VectorSubcoreMesh(core_axis_name="c", subcore_axis_name="s",
                                  num_cores=1)
    nrow = bt + 8
    q = 15632

    @functools.partial(
        pl.kernel,
        out_type=jax.ShapeDtypeStruct((nrow,), jnp.int32),
        mesh=mesh,
        scratch_types=[
            pltpu.VMEM((_WC,), jnp.int32),
            pltpu.VMEM((_WC,), jnp.int32),
            pltpu.VMEM((_WC,), jnp.int32),
            pltpu.VMEM_SHARED((nrow,), jnp.int32),
            pltpu.SemaphoreType.DMA,
        ],
    )
    def k(widx_hbm, ptr_hbm, idxb, keyb, gatb, shared, sem):
        wid = lax.axis_index("s")
        iota = lax.iota(jnp.int32, 16)

        ibase = jnp.minimum(wid * q, nrow - q)

        def istep(j, _):
            gatb[pl.ds(16 * j, 16)] = ibase + 16 * j + iota + 1
            return 0
        lax.fori_loop(0, q // 16, istep, 0)
        pltpu.sync_copy(gatb.at[pl.ds(0, q)], shared.at[pl.ds(ibase, q)])

        wbase = wid * _WC
        pltpu.sync_copy(widx_hbm.at[pl.ds(wbase, _WC)], idxb)

        def kstep(j, _):
            keyb[pl.ds(16 * j, 16)] = -(wbase + 16 * j + iota)
            return 0
        lax.fori_loop(0, _WC // 16, kstep, 0)

        plsc.subcore_barrier()
        pltpu.sync_copy(keyb, shared.at[idxb])
        plsc.subcore_barrier()

        def one_pass(p, _):
            pltpu.sync_copy(shared.at[idxb], gatb)

            def fstep(j, _):
                sl = pl.ds(16 * j, 16)
                gatb[sl] = jnp.where(keyb[sl] < gatb[sl], idxb[sl], bt)
                return 0
            lax.fori_loop(0, _WC // 16, fstep, 0)
            pltpu.sync_copy(keyb, shared.at[gatb])
            plsc.subcore_barrier()
            return 0
        lax.fori_loop(0, _WP, one_pass, 0)

        pltpu.sync_copy(shared.at[pl.ds(ibase, q)], gatb.at[pl.ds(0, q)])
        pltpu.sync_copy(gatb.at[pl.ds(0, q)], ptr_hbm.at[pl.ds(ibase, q)])

    return k(widx_pad)


_DC = 256
_DL = 32


def _dest(vtab, keytab, e, bt):
    mesh = plsc.VectorSubcoreMesh(core_axis_name="c", subcore_axis_name="s")
    cmax = bt // _DC
    lastbase = bt - _DC

    @functools.partial(
        pl.kernel,
        out_type=jax.ShapeDtypeStruct((bt, _D), jnp.float32),
        mesh=mesh,
        scratch_types=[
            pltpu.VMEM((_DC,), jnp.int32),
            pltpu.VMEM((_DC,), jnp.int32),
            pltpu.VMEM((_DC,), jnp.int32),
            pltpu.VMEM((_DC,), jnp.int32),
            pltpu.VMEM((_DC, _D), jnp.float32),
            pltpu.VMEM((_DC, _D), jnp.float32),
            pltpu.SemaphoreType.DMA,
            pltpu.SemaphoreType.DMA,
        ],
    )
    def k(v_hbm, key_hbm, out_hbm, ptr0, vr0, ptr1, vr1, buf0, buf1,
          sem0, sem1):
        wid = lax.axis_index("s") * 2 + lax.axis_index("c")
        iota16 = lax.iota(jnp.int32, 16)
        del iota16

        def base_of(l):
            c = jnp.minimum(wid + _NW * l, cmax)
            return jnp.minimum(c * _DC, lastbase)

        def load_start(l, ptrb, vrb, bufb, semb):
            bs = base_of(l)
            pltpu.sync_copy(key_hbm.at[pl.ds(bs, _DC)], ptrb)

            def vstep(i, _):
                sl = pl.ds(16 * i, 16)
                key = ptrb[sl]
                vrb[sl] = jnp.where(key <= 0, -key, e + key - 1)
                return 0
            lax.fori_loop(0, _DC // 16, vstep, 0)
            pltpu.async_copy(v_hbm.at[vrb], bufb, semb)

        bufs = ((ptr0, vr0, buf0, sem0), (ptr1, vr1, buf1, sem1))
        load_start(0, *bufs[0])
        load_start(1, *bufs[1])

        def pair(j, carry):
            for bb, (ptrb, vrb, bufb, semb) in enumerate(bufs):
                l = 2 * j + bb
                pltpu.make_async_copy(v_hbm.at[vrb], bufb, semb).wait()
                pltpu.sync_copy(bufb, out_hbm.at[pl.ds(base_of(l), _DC)])

                @pl.when(l + 2 < _DL)
                def _():
                    load_start(l + 2, ptrb, vrb, bufb, semb)
            return carry

        lax.fori_loop(0, _DL // 2, pair, 0)

    return k(vtab, keytab)


def kernel(expressions_encodings, symbols_encodings, expr_idx, token_idx,
           symbol_idx, Wu, bu, Wg, bg):
    b, t, d = expressions_encodings.shape
    e = expr_idx.shape[0]
    flat = expressions_encodings.reshape(b * t, d)
    flat_idx = (t * expr_idx + token_idx).astype(jnp.int32)
    fidx_pad = jnp.pad(flat_idx, (0, _EPAD - e))
    sidx_pad = jnp.pad(symbol_idx.astype(jnp.int32), (0, _EPAD - e))
    occ_p, sym_p = _gather_two(flat, symbols_encodings, fidx_pad, sidx_pad)
    bt = b * t
    vtab = _build_v(flat, occ_p, sym_p, Wu, bu, Wg[:d], Wg[d:], bg, e, bt)
    widx_pad = jnp.concatenate([flat_idx, jnp.full((_EPAD - e,), bt, jnp.int32)])
    keytab = _winner(widx_pad, bt, e)
    out = _dest(vtab, keytab, e, bt)
    return out.reshape(b, t, d)


# speedup vs baseline: 329.9797x; 1.6414x over previous
import functools

import jax
import jax.numpy as jnp
from jax import lax
from jax.experimental import pallas as pl
from jax.experimental.pallas import tpu as pltpu
from jax.experimental.pallas import tpu_sc as plsc

_D = 128
_NW = 32
_GC = 480
_GS = 20
_EPAD = _NW * _GC * _GS

_BLK = 2000


def _gate_body(occ_ref, sym_ref, wu_ref, bu_ref, wg1_ref, wg2_ref, bg_ref, out_ref):
    occ = occ_ref[...]
    sym = sym_ref[...]
    u = jnp.dot(sym, wu_ref[...], preferred_element_type=jnp.float32) + bu_ref[...]
    u = jnp.maximum(u, 0.0)
    z = (jnp.dot(occ, wg1_ref[...], preferred_element_type=jnp.float32)
         + jnp.dot(u, wg2_ref[...], preferred_element_type=jnp.float32)
         + bg_ref[...])
    g = jax.nn.sigmoid(z)
    out_ref[...] = g * occ + (1.0 - g) * u


def _copy_body(flat_ref, out_ref):
    out_ref[...] = flat_ref[...]


def _gate_alias_body(occ_ref, sym_ref, wu_ref, bu_ref, wg1_ref, wg2_ref,
                     bg_ref, v0_ref, out_ref):
    del v0_ref
    _gate_body(occ_ref, sym_ref, wu_ref, bu_ref, wg1_ref, wg2_ref, bg_ref,
               out_ref)


def _build_v(flat, occ, sym, Wu, bu, Wg1, Wg2, bg, e, bt):
    d = _D
    nv = e + bt
    row_spec = pl.BlockSpec((_BLK, d), lambda i: (i, 0))
    full_spec = pl.BlockSpec((d, d), lambda i: (0, 0))
    bias_spec = pl.BlockSpec((1, d), lambda i: (0, 0))
    ecols = e // _BLK
    v0 = pl.pallas_call(
        _copy_body,
        grid=(bt // _BLK,),
        in_specs=[row_spec],
        out_specs=pl.BlockSpec((_BLK, d), lambda i, ecols=ecols: (i + ecols, 0)),
        out_shape=jax.ShapeDtypeStruct((nv, d), jnp.float32),
    )(flat)
    return pl.pallas_call(
        _gate_alias_body,
        grid=(ecols,),
        in_specs=[row_spec, row_spec, full_spec, bias_spec, full_spec,
                  full_spec, bias_spec,
                  pl.BlockSpec(memory_space=pl.ANY)],
        out_specs=row_spec,
        out_shape=jax.ShapeDtypeStruct((nv, d), jnp.float32),
        input_output_aliases={7: 0},
    )(occ, sym, Wu, bu.reshape(1, d), Wg1, Wg2, bg.reshape(1, d), v0)


def _gather_two(flat, symbols, fidx_pad, sidx_pad):
    mesh = plsc.VectorSubcoreMesh(core_axis_name="c", subcore_axis_name="s")

    @functools.partial(
        pl.kernel,
        out_type=(jax.ShapeDtypeStruct((_EPAD, _D), jnp.float32),
                  jax.ShapeDtypeStruct((_EPAD, _D), jnp.float32)),
        mesh=mesh,
        scratch_types=[
            pltpu.VMEM((_GC,), jnp.int32),
            pltpu.VMEM((_GC,), jnp.int32),
            pltpu.VMEM((_GC, _D), jnp.float32),
            pltpu.VMEM((_GC, _D), jnp.float32),
            pltpu.SemaphoreType.DMA,
            pltpu.SemaphoreType.DMA,
        ],
    )
    def k(flat_hbm, sym_hbm, fidx_hbm, sidx_hbm, occ_hbm, symo_hbm,
          idx0, idx1, buf0, buf1, sem0, sem1):
        wid = lax.axis_index("s") * 2 + lax.axis_index("c")
        base = wid * (_GC * _GS)
        for tbl, ih, oh in ((flat_hbm, fidx_hbm, occ_hbm),
                            (sym_hbm, sidx_hbm, symo_hbm)):
            pltpu.sync_copy(ih.at[pl.ds(base, _GC)], idx0)
            pltpu.async_copy(tbl.at[idx0], buf0, sem0)
            pltpu.sync_copy(ih.at[pl.ds(base + _GC, _GC)], idx1)
            pltpu.async_copy(tbl.at[idx1], buf1, sem1)

            def pair(g, carry, tbl=tbl, ih=ih, oh=oh):
                for b, (idxb, bufb, semb) in enumerate(
                        ((idx0, buf0, sem0), (idx1, buf1, sem1))):
                    s = 2 * g + b
                    pltpu.make_async_copy(tbl.at[idxb], bufb, semb).wait()
                    pltpu.sync_copy(bufb, oh.at[pl.ds(base + s * _GC, _GC)])

                    @pl.when(s + 2 < _GS)
                    def _():
                        pltpu.sync_copy(ih.at[pl.ds(base + (s + 2) * _GC, _GC)],
                                        idxb)
                        pltpu.async_copy(tbl.at[idxb], bufb, semb)
                return carry

            lax.fori_loop(0, _GS // 2, pair, 0)

    return k(flat, symbols, fidx_pad, sidx_pad)


_WC = 19200
_WP = 12


def _winner(widx_pad, bt, e):
    mesh = plsc.VectorSubcoreMesh(core_axis_name="c", subcore_axis_name="s",
                                  num_cores=1)
    nrow = bt + 8
    q = 15632

    @functools.partial(
        pl.kernel,
        out_type=jax.ShapeDtypeStruct((nrow,), jnp.int32),
        mesh=mesh,
        scratch_types=[
            pltpu.VMEM((_WC,), jnp.int32),
            pltpu.VMEM((_WC,), jnp.int32),
            pltpu.VMEM((_WC,), jnp.int32),
            pltpu.VMEM((16,), jnp.int32),
            pltpu.VMEM_SHARED((nrow,), jnp.int32),
            pltpu.SemaphoreType.DMA,
        ],
        compiler_params=pltpu.CompilerParams(needs_layout_passes=False),
    )
    def k(widx_hbm, ptr_hbm, idxb, keyb, gatb, accb, shared, sem):
        wid = lax.axis_index("s")
        iota = lax.iota(jnp.int32, 16)

        ibase = jnp.minimum(wid * q, nrow - q)

        def istep(j, _):
            gatb[pl.ds(16 * j, 16)] = ibase + 16 * j + iota + 1
            return 0
        lax.fori_loop(0, q // 16, istep, 0)
        pltpu.sync_copy(gatb.at[pl.ds(0, q)], shared.at[pl.ds(ibase, q)])

        wbase = wid * _WC
        pltpu.sync_copy(widx_hbm.at[pl.ds(wbase, _WC)], idxb)

        def kstep(j, _):
            keyb[pl.ds(16 * j, 16)] = -(wbase + 16 * j + iota)
            return 0
        lax.fori_loop(0, _WC // 16, kstep, 0)

        accb[pl.ds(0, 16)] = jnp.zeros((16,), jnp.int32) + 1

        plsc.subcore_barrier()
        pltpu.sync_copy(keyb, shared.at[idxb])
        plsc.subcore_barrier()

        def one_pass(p, _):
            act = accb[pl.ds(0, 16)][0]
            accb[pl.ds(0, 16)] = jnp.zeros((16,), jnp.int32)

            @pl.when(act > 0)
            def _():
                pltpu.sync_copy(shared.at[idxb], gatb)

                def fstep(j, _):
                    sl = pl.ds(16 * j, 16)
                    m = keyb[sl] < gatb[sl]
                    gatb[sl] = jnp.where(m, idxb[sl], bt)
                    accb[pl.ds(0, 16)] = (
                        accb[pl.ds(0, 16)]
                        + plsc.all_reduce_population_count(m))
                    return 0
                lax.fori_loop(0, _WC // 16, fstep, 0)
                pltpu.sync_copy(keyb, shared.at[gatb])

            plsc.subcore_barrier()
            return 0
        lax.fori_loop(0, _WP, one_pass, 0)

        pltpu.sync_copy(shared.at[pl.ds(ibase, q)], gatb.at[pl.ds(0, q)])
        pltpu.sync_copy(gatb.at[pl.ds(0, q)], ptr_hbm.at[pl.ds(ibase, q)])

    return k(widx_pad)


_DC = 256
_DL = 32


def _dest(vtab, keytab, e, bt):
    mesh = plsc.VectorSubcoreMesh(core_axis_name="c", subcore_axis_name="s")
    cmax = bt // _DC
    lastbase = bt - _DC

    @functools.partial(
        pl.kernel,
        out_type=jax.ShapeDtypeStruct((bt, _D), jnp.float32),
        mesh=mesh,
        scratch_types=[
            pltpu.VMEM((_DC,), jnp.int32),
            pltpu.VMEM((_DC,), jnp.int32),
            pltpu.VMEM((_DC,), jnp.int32),
            pltpu.VMEM((_DC,), jnp.int32),
            pltpu.VMEM((_DC, _D), jnp.float32),
            pltpu.VMEM((_DC, _D), jnp.float32),
            pltpu.SemaphoreType.DMA,
            pltpu.SemaphoreType.DMA,
        ],
    )
    def k(v_hbm, key_hbm, out_hbm, ptr0, vr0, ptr1, vr1, buf0, buf1,
          sem0, sem1):
        wid = lax.axis_index("s") * 2 + lax.axis_index("c")
        iota16 = lax.iota(jnp.int32, 16)
        del iota16

        def base_of(l):
            c = jnp.minimum(wid + _NW * l, cmax)
            return jnp.minimum(c * _DC, lastbase)

        def load_start(l, ptrb, vrb, bufb, semb):
            bs = base_of(l)
            pltpu.sync_copy(key_hbm.at[pl.ds(bs, _DC)], ptrb)

            def vstep(i, _):
                sl = pl.ds(16 * i, 16)
                key = ptrb[sl]
                vrb[sl] = jnp.where(key <= 0, -key, e + key - 1)
                return 0
            lax.fori_loop(0, _DC // 16, vstep, 0)
            pltpu.async_copy(v_hbm.at[vrb], bufb, semb)

        bufs = ((ptr0, vr0, buf0, sem0), (ptr1, vr1, buf1, sem1))
        load_start(0, *bufs[0])
        load_start(1, *bufs[1])

        def pair(j, carry):
            for bb, (ptrb, vrb, bufb, semb) in enumerate(bufs):
                l = 2 * j + bb
                pltpu.make_async_copy(v_hbm.at[vrb], bufb, semb).wait()
                pltpu.sync_copy(bufb, out_hbm.at[pl.ds(base_of(l), _DC)])

                @pl.when(l + 2 < _DL)
                def _():
                    load_start(l + 2, ptrb, vrb, bufb, semb)
            return carry

        lax.fori_loop(0, _DL // 2, pair, 0)

    return k(vtab, keytab)


def kernel(expressions_encodings, symbols_encodings, expr_idx, token_idx,
           symbol_idx, Wu, bu, Wg, bg):
    b, t, d = expressions_encodings.shape
    e = expr_idx.shape[0]
    flat = expressions_encodings.reshape(b * t, d)
    flat_idx = (t * expr_idx + token_idx).astype(jnp.int32)
    fidx_pad = jnp.pad(flat_idx, (0, _EPAD - e))
    sidx_pad = jnp.pad(symbol_idx.astype(jnp.int32), (0, _EPAD - e))
    occ_p, sym_p = _gather_two(flat, symbols_encodings, fidx_pad, sidx_pad)
    bt = b * t
    vtab = _build_v(flat, occ_p, sym_p, Wu, bu, Wg[:d], Wg[d:], bg, e, bt)
    widx_pad = jnp.concatenate([flat_idx, jnp.full((_EPAD - e,), bt, jnp.int32)])
    keytab = _winner(widx_pad, bt, e)
    out = _dest(vtab, keytab, e, bt)
    return out.reshape(b, t, d)
